# Initial kernel scaffold; baseline (speedup 1.0000x reference)
import jax
import jax.numpy as jnp
from jax.experimental import pallas as pl


def kernel(input, pe):
    raise NotImplementedError("write your pallas kernel here")


# speedup vs baseline: 1.9545x; 1.9545x over previous
import math

import jax
import jax.numpy as jnp
from jax.experimental import pallas as pl


def _pe_add_kernel(x_ref, pe_ref, o_ref, *, scale):
    o_ref[...] = x_ref[...] * scale + pe_ref[...][None, :, :]


def kernel(input, pe):
    batch, seq, d_model = input.shape
    scale = math.sqrt(pe.shape[1])
    blk = 512
    grid = (seq // blk,)
    return pl.pallas_call(
        lambda x_ref, pe_ref, o_ref: _pe_add_kernel(x_ref, pe_ref, o_ref, scale=scale),
        grid=grid,
        in_specs=[
            pl.BlockSpec((batch, blk, d_model), lambda i: (0, i, 0)),
            pl.BlockSpec((blk, d_model), lambda i: (i, 0)),
        ],
        out_specs=pl.BlockSpec((batch, blk, d_model), lambda i: (0, i, 0)),
        out_shape=jax.ShapeDtypeStruct((batch, seq, d_model), input.dtype),
    )(input, pe)
